# Initial kernel scaffold; baseline (speedup 1.0000x reference)
import jax
import jax.numpy as jnp
from jax.experimental import pallas as pl


def kernel(class_feature, features, q_gamma, q_beta, Wq, kv_gamma, kv_beta, Wkv, proj_W, proj_b):
    raise NotImplementedError("write your pallas kernel here")


# speedup vs baseline: 2.8029x; 2.8029x over previous
import jax
import jax.numpy as jnp
from jax.experimental import pallas as pl
from jax.experimental.pallas import tpu as pltpu

_H = 8
_EPS = 1e-5
_BN = 512


def kernel(class_feature, features, q_gamma, q_beta, Wq, kv_gamma, kv_beta, Wkv, proj_W, proj_b):
    B, N, C = features.shape
    H = _H
    D = C // H
    BN = _BN
    NC = N // BN
    scale = C ** -0.5

    cf2 = class_feature.reshape(1, C)
    qg2 = q_gamma.reshape(1, C)
    qb2 = q_beta.reshape(1, C)
    kvg2 = kv_gamma.reshape(1, C)
    kvb2 = kv_beta.reshape(1, C)
    pb2 = proj_b.reshape(1, C)
    Wk = Wkv[:C]
    Wv = Wkv[C:]

    def body(x_ref, cf_ref, qg_ref, qb_ref, wq_ref, wk_ref, wv_ref,
             kvg_ref, kvb_ref, pw_ref, pb_ref, o_ref,
             waug, acc, m_s, d_s, t_s, g0_s, g1_s):
        nc = pl.program_id(1)

        head_mask = jnp.where(
            jax.lax.broadcasted_iota(jnp.int32, (H, C), 1) // D
            == jax.lax.broadcasted_iota(jnp.int32, (H, C), 0),
            1.0, 0.0).astype(jnp.float32)

        @pl.when(nc == 0)
        def _prep():
            cf = cf_ref[...]
            mu = jnp.mean(cf, axis=1, keepdims=True)
            xc = cf - mu
            var = jnp.mean(xc * xc, axis=1, keepdims=True)
            ln = xc * jax.lax.rsqrt(var + _EPS) * qg_ref[...] + qb_ref[...]
            q = jax.lax.dot_general(ln, wq_ref[...], (((1,), (1,)), ((), ())),
                                    preferred_element_type=jnp.float32)
            qs = q * scale
            A = jnp.broadcast_to(qs, (H, C)) * head_mask
            wl = jax.lax.dot_general(A, wk_ref[...], (((1,), (0,)), ((), ())),
                                     preferred_element_type=jnp.float32)
            wlg = wl * kvg_ref[...]
            waug[0:H, :] = wlg
            row = jax.lax.broadcasted_iota(jnp.int32, (8, C), 0)
            waug[H:H + 8, :] = jnp.where(row == 0, 1.0, 0.0)
            g1_s[...] = jnp.sum(wlg, axis=1, keepdims=True)
            g0_s[...] = jnp.sum(wl * kvb_ref[...], axis=1, keepdims=True)
            m_s[...] = jnp.full((H, 1), -1e30, jnp.float32)
            d_s[...] = jnp.zeros((H, 1), jnp.float32)
            t_s[...] = jnp.zeros((H, 1), jnp.float32)
            acc[...] = jnp.zeros((H, C), jnp.float32)

        x = x_ref[0]
        m1 = jax.lax.dot_general(waug[...], x, (((1,), (1,)), ((), ())),
                                 preferred_element_type=jnp.float32)
        ones_row = jnp.ones((1, C), jnp.float32)
        m2 = jax.lax.dot_general(ones_row, x * x, (((1,), (1,)), ((), ())),
                                 preferred_element_type=jnp.float32)

        inv_c = 1.0 / C
        mu_r = m1[H:H + 1, :] * inv_c
        var_r = m2 * inv_c - mu_r * mu_r
        s_r = jax.lax.rsqrt(var_r + _EPS)
        logits = s_r * (m1[0:H, :] - mu_r * g1_s[...]) + g0_s[...]

        m_prev = m_s[...]
        lm = jnp.max(logits, axis=1, keepdims=True)
        m_new = jnp.maximum(m_prev, lm)
        alpha = jnp.exp(m_prev - m_new)
        p = jnp.exp(logits - m_new)
        ps = p * s_r
        d_s[...] = d_s[...] * alpha + jnp.sum(p, axis=1, keepdims=True)
        t_s[...] = t_s[...] * alpha + jnp.sum(ps * mu_r, axis=1, keepdims=True)
        m_s[...] = m_new
        acc[...] = acc[...] * alpha + jax.lax.dot_general(
            ps, x, (((1,), (0,)), ((), ())), preferred_element_type=jnp.float32)

        @pl.when(nc == NC - 1)
        def _fin():
            dinv = 1.0 / d_s[...]
            S = kvg_ref[...] * (acc[...] * dinv - t_s[...] * dinv) + kvb_ref[...]
            R = jax.lax.dot_general(S, wv_ref[...], (((1,), (1,)), ((), ())),
                                    preferred_element_type=jnp.float32)
            agg = jnp.sum(R * head_mask, axis=0, keepdims=True)
            o = jax.lax.dot_general(agg, pw_ref[...], (((1,), (1,)), ((), ())),
                                    preferred_element_type=jnp.float32)
            o_ref[...] = (cf_ref[...] + o + pb_ref[...]).reshape(1, 1, C)

    full = lambda shape: pl.BlockSpec(shape, lambda b, nc: tuple(0 for _ in shape))
    out = pl.pallas_call(
        body,
        grid=(B, NC),
        in_specs=[
            pl.BlockSpec((1, BN, C), lambda b, nc: (b, nc, 0)),
            full((1, C)), full((1, C)), full((1, C)),
            full((C, C)), full((C, C)), full((C, C)),
            full((1, C)), full((1, C)),
            full((C, C)), full((1, C)),
        ],
        out_specs=pl.BlockSpec((1, 1, C), lambda b, nc: (b, 0, 0)),
        out_shape=jax.ShapeDtypeStruct((B, 1, C), jnp.float32),
        scratch_shapes=[
            pltpu.VMEM((2 * H, C), jnp.float32),
            pltpu.VMEM((H, C), jnp.float32),
            pltpu.VMEM((H, 1), jnp.float32),
            pltpu.VMEM((H, 1), jnp.float32),
            pltpu.VMEM((H, 1), jnp.float32),
            pltpu.VMEM((H, 1), jnp.float32),
            pltpu.VMEM((H, 1), jnp.float32),
        ],
        compiler_params=pltpu.CompilerParams(
            dimension_semantics=("parallel", "arbitrary"),
        ),
    )(features, cf2, qg2, qb2, Wq, Wk, Wv, kvg2, kvb2, proj_W, pb2)
    return out
